# baseline (speedup 1.0000x reference)
import functools

import jax
import jax.numpy as jnp
from jax import lax
from jax.experimental import pallas as pl
from jax.experimental.pallas import tpu as pltpu
from jax.experimental.pallas import tpu_sc as plsc

T = 200
D = 64
B = 4096
V = 100000

NC = 2
NS = 16
NW = NC * NS
BBLK = B // NW
LANES = 16
NBUF = 3


def _body(x_hbm, tok_hbm, pos_hbm, out_hbm, idx_v, pos_v,
          g0, g1, g2, c0, c1, c2, gs0, gs1, gs2, ss0, ss1, ss2):
    gbufs = (g0, g1, g2)
    cbufs = (c0, c1, c2)
    gsems = (gs0, gs1, gs2)
    ssems = (ss0, ss1, ss2)

    wid = lax.axis_index("s") * NC + lax.axis_index("c")

    pltpu.sync_copy(x_hbm.at[:, wid], idx_v)
    pltpu.sync_copy(pos_hbm, pos_v)

    rows_g = [(jax.lax.iota(jnp.int32, LANES) + g * LANES) for g in range(8)]

    def g_start(t, j):
        pltpu.make_async_copy(
            tok_hbm.at[idx_v.at[t // 8, t % 8]], gbufs[j], gsems[j]).start()

    def g_wait(t, j):
        pltpu.make_async_copy(
            tok_hbm.at[idx_v.at[t // 8, t % 8]], gbufs[j], gsems[j]).wait()

    def s_copy(t, j):
        return pltpu.make_async_copy(
            cbufs[j], out_hbm.at[t, :, wid], ssems[j])

    def compute(t, j):
        gbuf = gbufs[j]
        cbuf = cbufs[j]

        tvec = jnp.broadcast_to(jnp.int32(t), (LANES,))

        @plsc.parallel_loop(0, D)
        def _col(d):
            dvec = jnp.broadcast_to(jnp.int32(d), (LANES,))
            pvec = plsc.load_gather(pos_v, [tvec, dvec])
            for g in range(8):
                vals = plsc.load_gather(gbuf, [rows_g[g], dvec])
                cbuf[d // 8, d % 8, pl.ds(g * LANES, LANES)] = vals + pvec

    def step(t, j, prefetch, swait):
        if prefetch:
            g_start(t + 2, (j + 2) % NBUF)
        g_wait(t, j)
        if swait:
            s_copy(t - NBUF, j).wait()
        compute(t, j)
        s_copy(t, j).start()

    g_start(0, 0)
    g_start(1, 1)
    step(0, 0, True, False)
    step(1, 1, True, False)
    step(2, 2, True, False)

    @pl.loop(0, (T - 2 * NBUF + 1) // NBUF)
    def _main(i):
        t0 = NBUF + i * NBUF
        for jj in range(NBUF):
            step(t0 + jj, jj, True, True)

    step(T - 2, 0, False, True)
    step(T - 1, 1, False, True)
    for t, j in ((T - 3, 2), (T - 2, 0), (T - 1, 1)):
        s_copy(t, j).wait()


@functools.partial(jax.jit, static_argnames=())
def kernel(x, token_table, pos_table):
    x4 = (x.astype(jnp.int32).T
          .reshape(T // 8, 8, B // 128, 128).transpose(0, 2, 1, 3))
    tok_p = jnp.pad(token_table, ((0, 0), (0, 128 - D)))
    f = pl.kernel(
        _body,
        out_type=jax.ShapeDtypeStruct((T, D // 8, B // 128, 8, 128),
                                      jnp.float32),
        mesh=plsc.VectorSubcoreMesh(core_axis_name="c", subcore_axis_name="s"),
        compiler_params=pltpu.CompilerParams(use_tc_tiling_on_sc=True,
                                             needs_layout_passes=False),
        scratch_types=[
            pltpu.VMEM((T // 8, 8, BBLK), jnp.int32),
            pltpu.VMEM((T, D), jnp.float32),
        ] + [pltpu.VMEM((BBLK, 128), jnp.float32)] * NBUF
          + [pltpu.VMEM((D // 8, 8, BBLK), jnp.float32)] * NBUF
          + [pltpu.SemaphoreType.DMA] * (2 * NBUF),
    )
    out5 = f(x4, tok_p, pos_table)
    return out5.transpose(2, 4, 0, 1, 3).reshape(B, T, D)
